# baseline (speedup 1.0000x reference)
import functools

import jax
import jax.numpy as jnp
import numpy as np
from jax import lax
from jax.experimental import pallas as pl
from jax.experimental.pallas import tpu as pltpu
from jax.experimental.pallas import tpu_sc as plsc

_PI = 3.141592
_C = 1500.0
_T_DT = 2.5e-08
_T_MIN = 2.33e-05
_S_NUM = 512
_S_RAD = 0.11
_G_N = 512
_G_D = 0.15 / 512
_T_SAMPLES = 2168
_BATCH = 32
_NUM_BATCHES = _S_NUM // _BATCH

_NW = 32
_ROWS_PER_W = _G_N // _NW
_LANES = 16
_JV = _G_N // _LANES

_K1 = 1.0 / (_C * _T_DT)
_K2 = np.float32(-_T_MIN / _T_DT)
_LO2 = np.float32((200.0 - _K2) ** 2)
_HI2 = np.float32((2166.0 - _K2) ** 2)
_LO2B = np.int32(_LO2.view(np.int32))
_SPANB = np.uint32(int(_HI2.view(np.int32)) - int(_LO2B))
_HALF = 8


def _tables():
    phi = np.linspace(0.0, 2 * _PI, _S_NUM)
    sx = (_S_RAD * np.cos(phi + _PI)).astype(np.float32)
    sy = (_S_RAD * np.sin(phi + _PI)).astype(np.float32)
    g = (np.linspace(-_G_N / 2, _G_N / 2, _G_N) * _G_D).astype(np.float32)
    dx2 = ((g[None, :] - sx[:, None]) * _K1) ** 2
    dy2 = ((g[None, :] - sy[:, None]) * _K1) ** 2
    return dx2.astype(np.float32), dy2.astype(np.float32)


def _rsqrt_lut(dx2, dy2):
    r2min = float((dx2.min(1) + dy2.min(1)).min())
    r2max = float((dx2.max(1) + dy2.max(1)).max())
    bmin = int(np.float32(r2min).view(np.int32)) >> 23
    bmax = int(np.float32(r2max).view(np.int32)) >> 23
    base = bmin << 9
    n = (bmax - bmin + 1) << 9
    bits = ((np.arange(n, dtype=np.int64) + base) << 14) | (1 << 13)
    vals = bits.astype(np.uint32).view(np.float32)
    lut = (1.0 / np.sqrt(vals.astype(np.float64))).astype(np.float32)
    return lut, np.int32(base)


def _block_bounds(dx2, dy2):
    bounds = np.zeros((_NUM_BATCHES, 2 * _G_N), dtype=np.int32)
    for b in range(_NUM_BATCHES):
        s = slice(b * _BATCH, (b + 1) * _BATCH)
        r2 = dx2[s][:, :, None] + dy2[s][:, None, :]
        va = ((r2 >= _LO2) & (r2 <= _HI2)).any(0)
        vb = va.reshape(_G_N, _JV, _LANES).any(2)
        for i in range(_G_N):
            idxs = np.nonzero(vb[i])[0]
            if len(idxs):
                bounds[b, i] = idxs[0]
                bounds[b, _G_N + i] = idxs[-1] + 1
    return bounds


_DX2_NP, _DY2_NP = _tables()
_LUT_NP, _LUT_BASE = _rsqrt_lut(_DX2_NP, _DY2_NP)
_LUT_N = _LUT_NP.shape[0]
_BOUNDS_NP = _block_bounds(_DX2_NP, _DY2_NP)
_IOTA16 = np.arange(16, dtype=np.int32)


def _sc_body(sig_hbm, dx2_hbm, dy2_hbm, lut_hbm, bounds_hbm, out_hbm, sig_v,
             dy2_v, dx2_v, tile_v, lut_v, blo_v, bhi_v, sem):
    wid = lax.axis_index("c") * 16 + lax.axis_index("s")
    rowa = wid * _HALF
    rowb = rowa + _G_N // 2
    zeros16 = jnp.zeros((_LANES,), jnp.float32)
    pltpu.sync_copy(lut_hbm, lut_v)

    def fire_or_drain(b, half0, fire):
        s0 = b * _BATCH

        def go(k, c2):
            for h0, r0 in ((half0, rowa), (half0 + _HALF, rowb)):
                cp = pltpu.make_async_copy(
                    tile_v.at[pl.ds(h0, _HALF)],
                    out_hbm.at[s0 + k, pl.ds(r0, _HALF)], sem)
                if fire:
                    cp.start()
                else:
                    cp.wait()
            return c2
        lax.fori_loop(0, _BATCH, go, 0)

    def batch_body(b, carry):
        s0 = b * _BATCH
        half0 = lax.rem(b, 2) * _ROWS_PER_W

        @pl.when(b >= 2)
        def _drain_prev():
            fire_or_drain(b - 2, half0, fire=False)

        pltpu.sync_copy(sig_hbm.at[pl.ds(s0, _BATCH)], sig_v)
        pltpu.sync_copy(dy2_hbm.at[pl.ds(s0, _BATCH)], dy2_v)
        pltpu.sync_copy(dx2_hbm.at[pl.ds(s0, _BATCH)], dx2_v)
        pltpu.sync_copy(bounds_hbm.at[b, pl.ds(rowa, _HALF)],
                        blo_v.at[pl.ds(0, _HALF)])
        pltpu.sync_copy(bounds_hbm.at[b, pl.ds(rowb, _HALF)],
                        blo_v.at[pl.ds(_HALF, _HALF)])
        pltpu.sync_copy(bounds_hbm.at[b, pl.ds(_G_N + rowa, _HALF)],
                        bhi_v.at[pl.ds(0, _HALF)])
        pltpu.sync_copy(bounds_hbm.at[b, pl.ds(_G_N + rowb, _HALF)],
                        bhi_v.at[pl.ds(_HALF, _HALF)])

        def zero_head(s, c2):
            sig_v[s, pl.ds(0, _LANES)] = zeros16
            return c2
        lax.fori_loop(0, _BATCH, zero_head, 0)

        def row_body(ii, c2):
            row = rowa + ii + (_G_N // 2 - _HALF) * lax.shift_right_logical(
                ii, 3)
            colv = jnp.full((_LANES,), row, jnp.int32)
            lane = lax.iota(jnp.int32, _LANES) == ii
            jlo = jnp.max(jnp.where(lane, blo_v[...], 0))
            jhi = jnp.max(jnp.where(lane, bhi_v[...], 0))

            def zero_blk(jv, c3):
                tile_v[half0 + ii, pl.ds(jv * _LANES, _LANES)] = zeros16
                return c3
            lax.fori_loop(0, jlo, zero_blk, 0)
            lax.fori_loop(jhi, _JV, zero_blk, 0)

            def jv_body(jv, c3):
                jbase = jv * _LANES
                acc = zeros16
                for s in range(_BATCH):
                    srow = jnp.full((_LANES,), s, jnp.int32)
                    dx2s = plsc.load_gather(dx2_v, [srow, colv])
                    dy2v = dy2_v[s, pl.ds(jbase, _LANES)]
                    r2 = dy2v + dx2s
                    ib = plsc.bitcast(r2, jnp.int32)
                    valid = lax.bitcast_convert_type(
                        ib - _LO2B, jnp.uint32) <= _SPANB
                    kidx = lax.shift_right_logical(ib, 14) - _LUT_BASE
                    yb = plsc.load_gather(lut_v, [kidx])
                    half = 0.5 * r2
                    yb = yb * (1.5 - half * yb * yb)
                    idx = r2 * yb + _K2
                    idxc = jnp.where(valid, idx, 0.0)
                    d0i = idxc.astype(jnp.int32)
                    wb = idxc - d0i.astype(jnp.float32)
                    y0 = plsc.load_gather(sig_v, [srow, d0i])
                    y1 = plsc.load_gather(sig_v, [srow, d0i + 1])
                    acc = acc + (y0 + wb * (y1 - y0))
                tile_v[half0 + ii, pl.ds(jbase, _LANES)] = acc
                return c3

            return lax.fori_loop(jlo, jhi, jv_body, c2)

        lax.fori_loop(0, _ROWS_PER_W, row_body, 0)
        fire_or_drain(b, half0, fire=True)
        return carry

    lax.fori_loop(0, _NUM_BATCHES, batch_body, 0)

    def drain_tail(b, carry):
        fire_or_drain(b, lax.rem(b, 2) * _ROWS_PER_W, fire=False)
        return carry

    lax.fori_loop(_NUM_BATCHES - 2, _NUM_BATCHES, drain_tail, 0)


@jax.jit
def kernel(x):
    sig = x[0]
    dx2 = jnp.asarray(_DX2_NP)
    dy2 = jnp.asarray(_DY2_NP)
    lut = jnp.asarray(_LUT_NP)
    bounds = jnp.asarray(_BOUNDS_NP)

    run = functools.partial(
        pl.kernel,
        out_type=jax.ShapeDtypeStruct((_S_NUM, _G_N, _G_N), jnp.float32),
        mesh=plsc.VectorSubcoreMesh(core_axis_name="c", subcore_axis_name="s"),
        scratch_types=[
            pltpu.VMEM((_BATCH, _T_SAMPLES), jnp.float32),
            pltpu.VMEM((_BATCH, _G_N), jnp.float32),
            pltpu.VMEM((_BATCH, _G_N), jnp.float32),
            pltpu.VMEM((2 * _ROWS_PER_W, _G_N), jnp.float32),
            pltpu.VMEM((_LUT_N,), jnp.float32),
            pltpu.VMEM((_ROWS_PER_W,), jnp.int32),
            pltpu.VMEM((_ROWS_PER_W,), jnp.int32),
            pltpu.SemaphoreType.DMA,
        ],
        compiler_params=pltpu.CompilerParams(
            use_tc_tiling_on_sc=False, needs_layout_passes=False),
    )(_sc_body)
    out = run(sig, dx2, dy2, lut, bounds)
    return out[None]


# speedup vs baseline: 1.2018x; 1.2018x over previous
import functools

import jax
import jax.numpy as jnp
import numpy as np
from jax import lax
from jax.experimental import pallas as pl
from jax.experimental.pallas import tpu as pltpu
from jax.experimental.pallas import tpu_sc as plsc

_PI = 3.141592
_C = 1500.0
_T_DT = 2.5e-08
_T_MIN = 2.33e-05
_S_NUM = 512
_S_RAD = 0.11
_G_N = 512
_G_D = 0.15 / 512
_T_SAMPLES = 2168
_BATCH = 32
_NUM_BATCHES = _S_NUM // _BATCH

_NW = 32
_ROWS_PER_W = _G_N // _NW
_LANES = 16
_JV = _G_N // _LANES

_K1 = 1.0 / (_C * _T_DT)
_K2 = np.float32(-_T_MIN / _T_DT)
_LO2 = np.float32((200.0 - _K2) ** 2)
_HI2 = np.float32((2166.0 - _K2) ** 2)
_LO2B = np.int32(_LO2.view(np.int32))
_SPANB = np.uint32(int(_HI2.view(np.int32)) - int(_LO2B))
_HALF = 8


def _tables():
    phi = np.linspace(0.0, 2 * _PI, _S_NUM)
    sx = (_S_RAD * np.cos(phi + _PI)).astype(np.float32)
    sy = (_S_RAD * np.sin(phi + _PI)).astype(np.float32)
    g = (np.linspace(-_G_N / 2, _G_N / 2, _G_N) * _G_D).astype(np.float32)
    dx2 = ((g[None, :] - sx[:, None]) * _K1) ** 2
    dy2 = ((g[None, :] - sy[:, None]) * _K1) ** 2
    return dx2.astype(np.float32), dy2.astype(np.float32)


def _rsqrt_lut(dx2, dy2):
    r2min = float((dx2.min(1) + dy2.min(1)).min())
    r2max = float((dx2.max(1) + dy2.max(1)).max())
    bmin = int(np.float32(r2min).view(np.int32)) >> 23
    bmax = int(np.float32(r2max).view(np.int32)) >> 23
    base = bmin << 9
    n = (bmax - bmin + 1) << 9
    bits = ((np.arange(n, dtype=np.int64) + base) << 14) | (1 << 13)
    vals = bits.astype(np.uint32).view(np.float32)
    lut = (1.0 / np.sqrt(vals.astype(np.float64))).astype(np.float32)
    return lut, np.int32(base)


def _block_bounds(dx2, dy2):
    bounds = np.zeros((_NUM_BATCHES, 2 * _G_N), dtype=np.int32)
    for b in range(_NUM_BATCHES):
        s = slice(b * _BATCH, (b + 1) * _BATCH)
        r2 = dx2[s][:, :, None] + dy2[s][:, None, :]
        va = ((r2 >= _LO2) & (r2 <= _HI2)).any(0)
        vb = va.reshape(_G_N, _JV, _LANES).any(2)
        for i in range(_G_N):
            idxs = np.nonzero(vb[i])[0]
            if len(idxs):
                bounds[b, i] = idxs[0]
                bounds[b, _G_N + i] = idxs[-1] + 1
    return bounds


_DX2_NP, _DY2_NP = _tables()
_LUT_NP, _LUT_BASE = _rsqrt_lut(_DX2_NP, _DY2_NP)
_LUT_N = _LUT_NP.shape[0]
_BOUNDS_NP = _block_bounds(_DX2_NP, _DY2_NP)
_IOTA16 = np.arange(16, dtype=np.int32)


def _sc_body(sig_hbm, dx2_hbm, dy2_hbm, lut_hbm, bounds_hbm, out_hbm, sig_v,
             dy2_v, dx2_v, tile_v, lut_v, blo_v, bhi_v, sem):
    wid = lax.axis_index("c") * 16 + lax.axis_index("s")
    rowa = wid * _HALF
    rowb = rowa + _G_N // 2
    zeros16 = jnp.zeros((_LANES,), jnp.float32)
    pltpu.sync_copy(lut_hbm, lut_v)

    def fire_or_drain(b, half0, fire):
        s0 = b * _BATCH

        def go(k, c2):
            for h0, r0 in ((half0, rowa), (half0 + _HALF, rowb)):
                cp = pltpu.make_async_copy(
                    tile_v.at[pl.ds(h0, _HALF)],
                    out_hbm.at[s0 + k, pl.ds(r0, _HALF)], sem)
                if fire:
                    cp.start()
                else:
                    cp.wait()
            return c2
        lax.fori_loop(0, _BATCH, go, 0)

    def batch_body(b, carry):
        s0 = b * _BATCH
        half0 = lax.rem(b, 2) * _ROWS_PER_W

        @pl.when(b >= 2)
        def _drain_prev():
            fire_or_drain(b - 2, half0, fire=False)

        pltpu.sync_copy(sig_hbm.at[pl.ds(s0, _BATCH)], sig_v)
        pltpu.sync_copy(dy2_hbm.at[pl.ds(s0, _BATCH)], dy2_v)
        pltpu.sync_copy(dx2_hbm.at[pl.ds(s0, _BATCH)], dx2_v)
        pltpu.sync_copy(bounds_hbm.at[b, pl.ds(rowa, _HALF)],
                        blo_v.at[pl.ds(0, _HALF)])
        pltpu.sync_copy(bounds_hbm.at[b, pl.ds(rowb, _HALF)],
                        blo_v.at[pl.ds(_HALF, _HALF)])
        pltpu.sync_copy(bounds_hbm.at[b, pl.ds(_G_N + rowa, _HALF)],
                        bhi_v.at[pl.ds(0, _HALF)])
        pltpu.sync_copy(bounds_hbm.at[b, pl.ds(_G_N + rowb, _HALF)],
                        bhi_v.at[pl.ds(_HALF, _HALF)])

        def zero_head(s, c2):
            sig_v[s, pl.ds(0, _LANES)] = zeros16
            return c2
        lax.fori_loop(0, _BATCH, zero_head, 0)

        def row_body(ii, c2):
            row = rowa + ii + (_G_N // 2 - _HALF) * lax.shift_right_logical(
                ii, 3)
            colv = jnp.full((_LANES,), row, jnp.int32)
            lane = lax.iota(jnp.int32, _LANES) == ii
            jlo = jnp.max(jnp.where(lane, blo_v[...], 0))
            jhi = jnp.max(jnp.where(lane, bhi_v[...], 0))

            def zero_blk(jv, c3):
                tile_v[half0 + ii, pl.ds(jv * _LANES, _LANES)] = zeros16
                return c3
            lax.fori_loop(0, jlo, zero_blk, 0)
            lax.fori_loop(jhi, _JV, zero_blk, 0)

            def jv_body(jv, c3):
                jbase = jv * _LANES
                acc = zeros16
                for s in range(_BATCH):
                    srow = jnp.full((_LANES,), s, jnp.int32)
                    dx2s = plsc.load_gather(dx2_v, [srow, colv])
                    dy2v = dy2_v[s, pl.ds(jbase, _LANES)]
                    r2 = dy2v + dx2s
                    valid = (r2 >= _LO2) & (r2 <= _HI2)
                    kidx = lax.shift_right_logical(
                        plsc.bitcast(r2, jnp.int32), 14) - _LUT_BASE
                    yb = plsc.load_gather(lut_v, [kidx])
                    half = 0.5 * r2
                    yb = yb * (1.5 - half * yb * yb)
                    idx = r2 * yb + _K2
                    idxc = jnp.where(valid, idx, 0.0)
                    d0i = idxc.astype(jnp.int32)
                    wb = idxc - d0i.astype(jnp.float32)
                    y0 = plsc.load_gather(sig_v, [srow, d0i])
                    y1 = plsc.load_gather(sig_v, [srow, d0i + 1])
                    acc = acc + (y0 + wb * (y1 - y0))
                tile_v[half0 + ii, pl.ds(jbase, _LANES)] = acc
                return c3

            return lax.fori_loop(jlo, jhi, jv_body, c2)

        lax.fori_loop(0, _ROWS_PER_W, row_body, 0)
        fire_or_drain(b, half0, fire=True)
        return carry

    lax.fori_loop(0, _NUM_BATCHES, batch_body, 0)

    def drain_tail(b, carry):
        fire_or_drain(b, lax.rem(b, 2) * _ROWS_PER_W, fire=False)
        return carry

    lax.fori_loop(_NUM_BATCHES - 2, _NUM_BATCHES, drain_tail, 0)


@jax.jit
def kernel(x):
    sig = x[0]
    dx2 = jnp.asarray(_DX2_NP)
    dy2 = jnp.asarray(_DY2_NP)
    lut = jnp.asarray(_LUT_NP)
    bounds = jnp.asarray(_BOUNDS_NP)

    run = functools.partial(
        pl.kernel,
        out_type=jax.ShapeDtypeStruct((_S_NUM, _G_N, _G_N), jnp.float32),
        mesh=plsc.VectorSubcoreMesh(core_axis_name="c", subcore_axis_name="s"),
        scratch_types=[
            pltpu.VMEM((_BATCH, _T_SAMPLES), jnp.float32),
            pltpu.VMEM((_BATCH, _G_N), jnp.float32),
            pltpu.VMEM((_BATCH, _G_N), jnp.float32),
            pltpu.VMEM((2 * _ROWS_PER_W, _G_N), jnp.float32),
            pltpu.VMEM((_LUT_N,), jnp.float32),
            pltpu.VMEM((_ROWS_PER_W,), jnp.int32),
            pltpu.VMEM((_ROWS_PER_W,), jnp.int32),
            pltpu.SemaphoreType.DMA,
        ],
        compiler_params=pltpu.CompilerParams(
            use_tc_tiling_on_sc=False, needs_layout_passes=False),
    )(_sc_body)
    out = run(sig, dx2, dy2, lut, bounds)
    return out[None]


# speedup vs baseline: 1.2038x; 1.0017x over previous
import functools

import jax
import jax.numpy as jnp
import numpy as np
from jax import lax
from jax.experimental import pallas as pl
from jax.experimental.pallas import tpu as pltpu
from jax.experimental.pallas import tpu_sc as plsc

_PI = 3.141592
_C = 1500.0
_T_DT = 2.5e-08
_T_MIN = 2.33e-05
_S_NUM = 512
_S_RAD = 0.11
_G_N = 512
_G_D = 0.15 / 512
_T_SAMPLES = 2168
_BATCH = 32
_NUM_BATCHES = _S_NUM // _BATCH

_NW = 32
_ROWS_PER_W = _G_N // _NW
_LANES = 16
_JV = _G_N // _LANES

_K1 = 1.0 / (_C * _T_DT)
_K2 = np.float32(-_T_MIN / _T_DT)
_LO2 = np.float32((200.0 - _K2) ** 2)
_HI2 = np.float32((2166.0 - _K2) ** 2)
_LO2B = np.int32(_LO2.view(np.int32))
_SPANB = np.uint32(int(_HI2.view(np.int32)) - int(_LO2B))
_HALF = 8


def _tables():
    phi = np.linspace(0.0, 2 * _PI, _S_NUM)
    sx = (_S_RAD * np.cos(phi + _PI)).astype(np.float32)
    sy = (_S_RAD * np.sin(phi + _PI)).astype(np.float32)
    g = (np.linspace(-_G_N / 2, _G_N / 2, _G_N) * _G_D).astype(np.float32)
    dx2 = ((g[None, :] - sx[:, None]) * _K1) ** 2
    dy2 = ((g[None, :] - sy[:, None]) * _K1) ** 2
    return dx2.astype(np.float32), dy2.astype(np.float32)


def _rsqrt_lut(dx2, dy2):
    r2min = float((dx2.min(1) + dy2.min(1)).min())
    r2max = float((dx2.max(1) + dy2.max(1)).max())
    bmin = int(np.float32(r2min).view(np.int32)) >> 23
    bmax = int(np.float32(r2max).view(np.int32)) >> 23
    base = bmin << 9
    n = (bmax - bmin + 1) << 9
    bits = ((np.arange(n, dtype=np.int64) + base) << 14) | (1 << 13)
    vals = bits.astype(np.uint32).view(np.float32)
    lut = (1.0 / np.sqrt(vals.astype(np.float64))).astype(np.float32)
    return lut, np.int32(base)


def _block_bounds(dx2, dy2):
    bounds = np.zeros((_NUM_BATCHES, 2 * _G_N), dtype=np.int32)
    for b in range(_NUM_BATCHES):
        s = slice(b * _BATCH, (b + 1) * _BATCH)
        r2 = dx2[s][:, :, None] + dy2[s][:, None, :]
        va = ((r2 >= _LO2) & (r2 <= _HI2)).any(0)
        vb = va.reshape(_G_N, _JV, _LANES).any(2)
        for i in range(_G_N):
            idxs = np.nonzero(vb[i])[0]
            if len(idxs):
                bounds[b, i] = idxs[0]
                bounds[b, _G_N + i] = idxs[-1] + 1
    return bounds


_DX2_NP, _DY2_NP = _tables()
_LUT_NP, _LUT_BASE = _rsqrt_lut(_DX2_NP, _DY2_NP)
_LUT_N = _LUT_NP.shape[0]
_BOUNDS_NP = _block_bounds(_DX2_NP, _DY2_NP)
_IOTA16 = np.arange(16, dtype=np.int32)


def _sc_body(sig_hbm, dx2_hbm, dy2_hbm, lut_hbm, bounds_hbm, out_hbm, sig_v,
             dy2_v, dx2_v, tile_v, lut_v, blo_v, bhi_v, sem):
    wid = lax.axis_index("c") * 16 + lax.axis_index("s")
    rowa = wid * _HALF
    rowb = rowa + _G_N // 2
    zeros16 = jnp.zeros((_LANES,), jnp.float32)
    pltpu.sync_copy(lut_hbm, lut_v)

    def fire_or_drain(b, half0, fire):
        s0 = b * _BATCH

        def go(k, c2):
            for h0, r0 in ((half0, rowa), (half0 + _HALF, rowb)):
                cp = pltpu.make_async_copy(
                    tile_v.at[pl.ds(h0, _HALF)],
                    out_hbm.at[0, s0 + k, pl.ds(r0, _HALF)], sem)
                if fire:
                    cp.start()
                else:
                    cp.wait()
            return c2
        lax.fori_loop(0, _BATCH, go, 0)

    def batch_body(b, carry):
        s0 = b * _BATCH
        half0 = lax.rem(b, 2) * _ROWS_PER_W

        @pl.when(b >= 2)
        def _drain_prev():
            fire_or_drain(b - 2, half0, fire=False)

        pltpu.sync_copy(sig_hbm.at[pl.ds(s0, _BATCH)], sig_v)
        pltpu.sync_copy(dy2_hbm.at[pl.ds(s0, _BATCH)], dy2_v)
        pltpu.sync_copy(dx2_hbm.at[pl.ds(s0, _BATCH)], dx2_v)
        pltpu.sync_copy(bounds_hbm.at[b, pl.ds(rowa, _HALF)],
                        blo_v.at[pl.ds(0, _HALF)])
        pltpu.sync_copy(bounds_hbm.at[b, pl.ds(rowb, _HALF)],
                        blo_v.at[pl.ds(_HALF, _HALF)])
        pltpu.sync_copy(bounds_hbm.at[b, pl.ds(_G_N + rowa, _HALF)],
                        bhi_v.at[pl.ds(0, _HALF)])
        pltpu.sync_copy(bounds_hbm.at[b, pl.ds(_G_N + rowb, _HALF)],
                        bhi_v.at[pl.ds(_HALF, _HALF)])

        def zero_head(s, c2):
            sig_v[s, pl.ds(0, _LANES)] = zeros16
            return c2
        lax.fori_loop(0, _BATCH, zero_head, 0)

        def row_body(ii, c2):
            row = rowa + ii + (_G_N // 2 - _HALF) * lax.shift_right_logical(
                ii, 3)
            colv = jnp.full((_LANES,), row, jnp.int32)
            lane = lax.iota(jnp.int32, _LANES) == ii
            jlo = jnp.max(jnp.where(lane, blo_v[...], 0))
            jhi = jnp.max(jnp.where(lane, bhi_v[...], 0))

            def zero_blk(jv, c3):
                tile_v[half0 + ii, pl.ds(jv * _LANES, _LANES)] = zeros16
                return c3
            lax.fori_loop(0, jlo, zero_blk, 0)
            lax.fori_loop(jhi, _JV, zero_blk, 0)

            def jv_body(jv, c3):
                jbase = jv * _LANES
                acc = zeros16
                for s in range(_BATCH):
                    srow = jnp.full((_LANES,), s, jnp.int32)
                    dx2s = plsc.load_gather(dx2_v, [srow, colv])
                    dy2v = dy2_v[s, pl.ds(jbase, _LANES)]
                    r2 = dy2v + dx2s
                    valid = (r2 >= _LO2) & (r2 <= _HI2)
                    kidx = lax.shift_right_logical(
                        plsc.bitcast(r2, jnp.int32), 14) - _LUT_BASE
                    yb = plsc.load_gather(lut_v, [kidx])
                    half = 0.5 * r2
                    yb = yb * (1.5 - half * yb * yb)
                    idx = r2 * yb + _K2
                    idxc = jnp.where(valid, idx, 0.0)
                    d0i = idxc.astype(jnp.int32)
                    wb = idxc - d0i.astype(jnp.float32)
                    y0 = plsc.load_gather(sig_v, [srow, d0i])
                    y1 = plsc.load_gather(sig_v, [srow, d0i + 1])
                    acc = acc + (y0 + wb * (y1 - y0))
                tile_v[half0 + ii, pl.ds(jbase, _LANES)] = acc
                return c3

            return lax.fori_loop(jlo, jhi, jv_body, c2)

        lax.fori_loop(0, _ROWS_PER_W, row_body, 0)
        fire_or_drain(b, half0, fire=True)
        return carry

    lax.fori_loop(0, _NUM_BATCHES, batch_body, 0)

    def drain_tail(b, carry):
        fire_or_drain(b, lax.rem(b, 2) * _ROWS_PER_W, fire=False)
        return carry

    lax.fori_loop(_NUM_BATCHES - 2, _NUM_BATCHES, drain_tail, 0)


@jax.jit
def kernel(x):
    sig = x[0]
    dx2 = jnp.asarray(_DX2_NP)
    dy2 = jnp.asarray(_DY2_NP)
    lut = jnp.asarray(_LUT_NP)
    bounds = jnp.asarray(_BOUNDS_NP)

    run = functools.partial(
        pl.kernel,
        out_type=jax.ShapeDtypeStruct((1, _S_NUM, _G_N, _G_N), jnp.float32),
        mesh=plsc.VectorSubcoreMesh(core_axis_name="c", subcore_axis_name="s"),
        scratch_types=[
            pltpu.VMEM((_BATCH, _T_SAMPLES), jnp.float32),
            pltpu.VMEM((_BATCH, _G_N), jnp.float32),
            pltpu.VMEM((_BATCH, _G_N), jnp.float32),
            pltpu.VMEM((2 * _ROWS_PER_W, _G_N), jnp.float32),
            pltpu.VMEM((_LUT_N,), jnp.float32),
            pltpu.VMEM((_ROWS_PER_W,), jnp.int32),
            pltpu.VMEM((_ROWS_PER_W,), jnp.int32),
            pltpu.SemaphoreType.DMA,
        ],
        compiler_params=pltpu.CompilerParams(
            use_tc_tiling_on_sc=False, needs_layout_passes=False),
    )(_sc_body)
    return run(sig, dx2, dy2, lut, bounds)


# speedup vs baseline: 1.4359x; 1.1928x over previous
import functools

import jax
import jax.numpy as jnp
import numpy as np
from jax import lax
from jax.experimental import pallas as pl
from jax.experimental.pallas import tpu as pltpu
from jax.experimental.pallas import tpu_sc as plsc

_PI = 3.141592
_C = 1500.0
_T_DT = 2.5e-08
_T_MIN = 2.33e-05
_S_NUM = 512
_S_RAD = 0.11
_G_N = 512
_G_D = 0.15 / 512
_T_SAMPLES = 2168
_BATCH = 32
_NUM_BATCHES = _S_NUM // _BATCH

_NW = 32
_ROWS_PER_W = _G_N // _NW
_LANES = 16
_JV = _G_N // _LANES

_K1 = 1.0 / (_C * _T_DT)
_K2 = np.float32(-_T_MIN / _T_DT)
_LO2 = np.float32((200.0 - _K2) ** 2)
_HI2 = np.float32((2166.0 - _K2) ** 2)
_LO2B = np.int32(_LO2.view(np.int32))
_SPANB = np.uint32(int(_HI2.view(np.int32)) - int(_LO2B))
_HALF = 8


def _tables():
    phi = np.linspace(0.0, 2 * _PI, _S_NUM)
    sx = (_S_RAD * np.cos(phi + _PI)).astype(np.float32)
    sy = (_S_RAD * np.sin(phi + _PI)).astype(np.float32)
    g = (np.linspace(-_G_N / 2, _G_N / 2, _G_N) * _G_D).astype(np.float32)
    dx2 = ((g[None, :] - sx[:, None]) * _K1) ** 2
    dy2 = ((g[None, :] - sy[:, None]) * _K1) ** 2
    return dx2.astype(np.float32), dy2.astype(np.float32)


def _rsqrt_lut(dx2, dy2):
    r2min = float((dx2.min(1) + dy2.min(1)).min())
    r2max = float((dx2.max(1) + dy2.max(1)).max())
    bmin = int(np.float32(r2min).view(np.int32)) >> 23
    bmax = int(np.float32(r2max).view(np.int32)) >> 23
    base = bmin << 9
    n = (bmax - bmin + 1) << 9
    bits = ((np.arange(n, dtype=np.int64) + base) << 14) | (1 << 13)
    vals = bits.astype(np.uint32).view(np.float32)
    lut = (1.0 / np.sqrt(vals.astype(np.float64))).astype(np.float32)
    return lut, np.int32(base)


def _block_bounds(dx2, dy2):
    bounds = np.zeros((_NUM_BATCHES, 2 * _G_N), dtype=np.int32)
    for b in range(_NUM_BATCHES):
        s = slice(b * _BATCH, (b + 1) * _BATCH)
        r2 = dx2[s][:, :, None] + dy2[s][:, None, :]
        va = ((r2 >= _LO2) & (r2 <= _HI2)).any(0)
        vb = va.reshape(_G_N, _JV, _LANES).any(2)
        for i in range(_G_N):
            idxs = np.nonzero(vb[i])[0]
            if len(idxs):
                bounds[b, i] = idxs[0]
                bounds[b, _G_N + i] = idxs[-1] + 1
    return bounds


_DX2_NP, _DY2_NP = _tables()
_LUT_NP, _LUT_BASE = _rsqrt_lut(_DX2_NP, _DY2_NP)
_LUT_N = _LUT_NP.shape[0]
_BOUNDS_NP = _block_bounds(_DX2_NP, _DY2_NP)
_IOTA16 = np.arange(16, dtype=np.int32)


def _sc_body(sig_hbm, dx2_hbm, dy2_hbm, lut_hbm, bounds_hbm, out_hbm, sig_v,
             dy2_v, dx2_v, tile_v, lut_v, blo_v, bhi_v, sem):
    wid = lax.axis_index("c") * 16 + lax.axis_index("s")
    rowa = wid * _HALF
    rowb = rowa + _G_N // 2
    zeros16 = jnp.zeros((_LANES,), jnp.float32)
    pltpu.sync_copy(lut_hbm, lut_v)

    def fire_or_drain(b, half0, fire):
        s0 = b * _BATCH

        def go(k, c2):
            for h0, r0 in ((half0, rowa), (half0 + _HALF, rowb)):
                cp = pltpu.make_async_copy(
                    tile_v.at[pl.ds(h0, _HALF)],
                    out_hbm.at[0, s0 + k, pl.ds(r0, _HALF)], sem)
                if fire:
                    cp.start()
                else:
                    cp.wait()
            return c2
        lax.fori_loop(0, _BATCH, go, 0)

    def batch_body(b, carry):
        s0 = b * _BATCH
        half0 = lax.rem(b, 2) * _ROWS_PER_W

        @pl.when(b >= 2)
        def _drain_prev():
            fire_or_drain(b - 2, half0, fire=False)

        pltpu.sync_copy(sig_hbm.at[pl.ds(s0, _BATCH)], sig_v)
        pltpu.sync_copy(dy2_hbm.at[pl.ds(s0, _BATCH)], dy2_v)
        pltpu.sync_copy(dx2_hbm.at[pl.ds(s0, _BATCH)], dx2_v)
        pltpu.sync_copy(bounds_hbm.at[b, pl.ds(rowa, _HALF)],
                        blo_v.at[pl.ds(0, _HALF)])
        pltpu.sync_copy(bounds_hbm.at[b, pl.ds(rowb, _HALF)],
                        blo_v.at[pl.ds(_HALF, _HALF)])
        pltpu.sync_copy(bounds_hbm.at[b, pl.ds(_G_N + rowa, _HALF)],
                        bhi_v.at[pl.ds(0, _HALF)])
        pltpu.sync_copy(bounds_hbm.at[b, pl.ds(_G_N + rowb, _HALF)],
                        bhi_v.at[pl.ds(_HALF, _HALF)])

        def zero_head(s, c2):
            sig_v[s, pl.ds(0, _LANES)] = zeros16
            return c2
        lax.fori_loop(0, _BATCH, zero_head, 0)

        def row_body(ii, c2):
            row = rowa + ii + (_G_N // 2 - _HALF) * lax.shift_right_logical(
                ii, 3)
            colv = jnp.full((_LANES,), row, jnp.int32)
            lane = lax.iota(jnp.int32, _LANES) == ii
            jlo = jnp.max(jnp.where(lane, blo_v[...], 0))
            jhi = jnp.max(jnp.where(lane, bhi_v[...], 0))

            def zero_blk(jv, c3):
                tile_v[half0 + ii, pl.ds(jv * _LANES, _LANES)] = zeros16
                return c3
            lax.fori_loop(0, jlo, zero_blk, 0)
            lax.fori_loop(jhi, _JV, zero_blk, 0)

            def jv_body(jv, c3):
                jbase = jv * _LANES
                acc = zeros16
                for s in range(_BATCH):
                    srow = jnp.full((_LANES,), s, jnp.int32)
                    dx2s = plsc.load_gather(dx2_v, [srow, colv])
                    dy2v = dy2_v[s, pl.ds(jbase, _LANES)]
                    r2 = dy2v + dx2s
                    valid = (r2 >= _LO2) & (r2 <= _HI2)
                    kidx = lax.shift_right_logical(
                        plsc.bitcast(r2, jnp.int32), 14) - _LUT_BASE
                    yb = plsc.load_gather(lut_v, [kidx])
                    half = 0.5 * r2
                    yb = yb * (1.5 - half * yb * yb)
                    idx = r2 * yb + _K2
                    idxc = jnp.where(valid, idx, 0.0)
                    d0i = idxc.astype(jnp.int32)
                    wb = idxc - d0i.astype(jnp.float32)
                    y0 = plsc.load_gather(sig_v, [srow, d0i])
                    y1 = plsc.load_gather(sig_v, [srow, d0i + 1])
                    acc = acc + (y0 + wb * (y1 - y0))
                tile_v[half0 + ii, pl.ds(jbase, _LANES)] = acc
                return c3

            return lax.fori_loop(jlo, jhi, jv_body, c2)

        lax.fori_loop(0, _ROWS_PER_W, row_body, 0)
        fire_or_drain(b, half0, fire=True)
        return carry

    lax.fori_loop(0, _NUM_BATCHES, batch_body, 0)

    def drain_tail(b, carry):
        fire_or_drain(b, lax.rem(b, 2) * _ROWS_PER_W, fire=False)
        return carry

    lax.fori_loop(_NUM_BATCHES - 2, _NUM_BATCHES, drain_tail, 0)


@jax.jit
def kernel(x):
    sig = x[0]
    dx2 = jnp.asarray(_DX2_NP)
    dy2 = jnp.asarray(_DY2_NP)
    lut = jnp.asarray(_LUT_NP)
    bounds = jnp.asarray(_BOUNDS_NP)

    run = functools.partial(
        pl.kernel,
        out_type=jax.ShapeDtypeStruct((1, _S_NUM, _G_N, _G_N), jnp.float32),
        mesh=plsc.VectorSubcoreMesh(core_axis_name="c", subcore_axis_name="s"),
        scratch_types=[
            pltpu.VMEM((_BATCH, _T_SAMPLES), jnp.float32),
            pltpu.VMEM((_BATCH, _G_N), jnp.float32),
            pltpu.VMEM((_BATCH, _G_N), jnp.float32),
            pltpu.VMEM((2 * _ROWS_PER_W, _G_N), jnp.float32),
            pltpu.VMEM((_LUT_N,), jnp.float32),
            pltpu.VMEM((_ROWS_PER_W,), jnp.int32),
            pltpu.VMEM((_ROWS_PER_W,), jnp.int32),
            pltpu.SemaphoreType.DMA,
        ],
        compiler_params=pltpu.CompilerParams(needs_layout_passes=False),
    )(_sc_body)
    return run(sig, dx2, dy2, lut, bounds)
